# baseline (speedup 1.0000x reference)
import functools

import jax
import jax.numpy as jnp
from jax import lax
from jax.experimental import pallas as pl
from jax.experimental.pallas import tpu as pltpu
from jax.experimental.pallas import tpu_sc as plsc

_N = 10000
_E = 320000
_D = 128
_H = 128
_OUT = 10
_G = 128

_NC = 2
_NS = 16
_HD = _D // _NC
_CHUNK = 80
_CH_PER_TEC = 256
_WCH = 32
_NWIN = _CH_PER_TEC // _WCH
_E_PER_TEC = _CHUNK * _CH_PER_TEC
_E_PAD = _E_PER_TEC * _NS
_ROWS = 10112
_ZROWS = _ROWS // _NS


_NBUF = 3


def _sc_agg_body(xs_hbm, srcs_hbm, dsts_hbm, zeros_hbm, out_hbm,
                 x_sh, acc_sh, src_v, dst_v,
                 rows0_v, rows1_v, rows2_v, sem0, sem1, sem2):
    c = lax.axis_index("c")
    s = lax.axis_index("s")
    cp_x = pltpu.async_copy(xs_hbm.at[c, pl.ds(s * _ZROWS, _ZROWS)],
                            x_sh.at[pl.ds(s * _ZROWS, _ZROWS)], sem0)
    cp_z = pltpu.async_copy(zeros_hbm, acc_sh.at[pl.ds(s * _ZROWS, _ZROWS)],
                            sem1)
    cp_x.wait()
    cp_z.wait()
    plsc.subcore_barrier()

    rows = (rows0_v, rows1_v, rows2_v)
    sems = (sem0, sem1, sem2)

    def window(w, carry):
        pltpu.sync_copy(srcs_hbm.at[s, pl.ds(w * _WCH, _WCH)], src_v)
        pltpu.sync_copy(dsts_hbm.at[s, pl.ds(w * _WCH, _WCH)], dst_v)
        for b in range(_NBUF):
            pltpu.async_copy(x_sh.at[src_v.at[b]], rows[b], sems[b])

        def step(g, cy):
            for b in range(_NBUF):
                j = g * _NBUF + b
                pltpu.make_async_copy(x_sh.at[src_v.at[j]], rows[b],
                                      sems[b]).wait()
                pltpu.sync_copy(rows[b], acc_sh.at[dst_v.at[j]], add=True)
                pltpu.async_copy(x_sh.at[src_v.at[j + _NBUF]], rows[b],
                                 sems[b])
            return cy

        turns = _WCH // _NBUF - 1
        lax.fori_loop(0, turns, step, carry, unroll=False)
        for j in range(turns * _NBUF, _WCH):
            b = j % _NBUF
            pltpu.make_async_copy(x_sh.at[src_v.at[j]], rows[b],
                                  sems[b]).wait()
            pltpu.sync_copy(rows[b], acc_sh.at[dst_v.at[j]], add=True)
            if j + _NBUF < _WCH:
                pltpu.async_copy(x_sh.at[src_v.at[j + _NBUF]], rows[b],
                                 sems[b])
        return carry

    lax.fori_loop(0, _NWIN, window, 0, unroll=False)
    plsc.subcore_barrier()
    pltpu.sync_copy(acc_sh.at[pl.ds(s * _ZROWS, _ZROWS)],
                    out_hbm.at[c, pl.ds(s * _ZROWS, _ZROWS)])


@functools.cache
def _build_sc_agg():
    return functools.partial(
        pl.kernel,
        out_type=jax.ShapeDtypeStruct((_NC, _ROWS, _HD), jnp.float32),
        mesh=plsc.VectorSubcoreMesh(core_axis_name="c", subcore_axis_name="s",
                                    num_cores=_NC, num_subcores=_NS),
        scratch_types=[
            pltpu.VMEM_SHARED((_ROWS, _HD), jnp.float32),
            pltpu.VMEM_SHARED((_ROWS, _HD), jnp.float32),
            pltpu.VMEM((_WCH, _CHUNK), jnp.int32),
            pltpu.VMEM((_WCH, _CHUNK), jnp.int32),
            pltpu.VMEM((_CHUNK, _HD), jnp.float32),
            pltpu.VMEM((_CHUNK, _HD), jnp.float32),
            pltpu.VMEM((_CHUNK, _HD), jnp.float32),
            pltpu.SemaphoreType.DMA,
            pltpu.SemaphoreType.DMA,
            pltpu.SemaphoreType.DMA,
        ],
    )(_sc_agg_body)


def _sc_agg(xs, srcs, dsts, zeros):
    return _build_sc_agg()(xs, srcs, dsts, zeros)


def _layer_body(xs_ref, a_ref, eps_ref, wa_ref, ba_ref, wb_ref, bb_ref,
                g_ref, be_ref, o_ref):
    x = jnp.concatenate([xs_ref[0, pl.ds(0, _N), :],
                         xs_ref[1, pl.ds(0, _N), :]], axis=1)
    agg = jnp.concatenate([a_ref[0, pl.ds(0, _N), :],
                           a_ref[1, pl.ds(0, _N), :]], axis=1)
    h = eps_ref[...] * x + agg
    h = jnp.maximum(jnp.dot(h, wa_ref[...], preferred_element_type=jnp.float32)
                    + ba_ref[...], 0.0)
    h = jnp.maximum(jnp.dot(h, wb_ref[...], preferred_element_type=jnp.float32)
                    + bb_ref[...], 0.0)
    mu = jnp.mean(h, axis=0)
    d = h - mu
    var = jnp.mean(d * d, axis=0)
    out = g_ref[...] * d * lax.rsqrt(var + 1e-5) + be_ref[...]
    o_ref[0, pl.ds(0, _N), :] = out[:, :_HD]
    o_ref[1, pl.ds(0, _N), :] = out[:, _HD:]


def _layer(xs, agg, eps, wa, ba, wb, bb, g, be):
    return pl.pallas_call(
        _layer_body,
        out_shape=jax.ShapeDtypeStruct((_NC, _ROWS, _HD), jnp.float32),
    )(xs, agg, jnp.reshape(1.0 + eps, (1, 1)), wa,
      jnp.reshape(ba, (1, _H)), wb, jnp.reshape(bb, (1, _H)),
      jnp.reshape(g, (1, _H)), jnp.reshape(be, (1, _H)))


def _layer3_head_body(xs_ref, a_ref, eps_ref, wa_ref, ba_ref, wb_ref, bb_ref,
                      g_ref, be_ref, bt_ref, l1w_ref, l1b_ref, l2w_ref,
                      l2b_ref, o_ref):
    x = jnp.concatenate([xs_ref[0, pl.ds(0, _N), :],
                         xs_ref[1, pl.ds(0, _N), :]], axis=1)
    agg = jnp.concatenate([a_ref[0, pl.ds(0, _N), :],
                           a_ref[1, pl.ds(0, _N), :]], axis=1)
    h = eps_ref[...] * x + agg
    h = jnp.maximum(jnp.dot(h, wa_ref[...], preferred_element_type=jnp.float32)
                    + ba_ref[...], 0.0)
    h = jnp.maximum(jnp.dot(h, wb_ref[...], preferred_element_type=jnp.float32)
                    + bb_ref[...], 0.0)
    mu = jnp.mean(h, axis=0)
    d = h - mu
    var = jnp.mean(d * d, axis=0)
    h = g_ref[...] * d * lax.rsqrt(var + 1e-5) + be_ref[...]
    onehot = (bt_ref[...][:, None]
              == lax.broadcasted_iota(jnp.int32, (1, _G), 1)).astype(jnp.float32)
    sums = lax.dot_general(onehot, h, (((0,), (0,)), ((), ())),
                           preferred_element_type=jnp.float32)
    cnt = jnp.sum(onehot, axis=0)
    pooled = sums / jnp.maximum(cnt, 1.0)[:, None]
    hh = jnp.maximum(jnp.dot(pooled, l1w_ref[...],
                             preferred_element_type=jnp.float32) + l1b_ref[...], 0.0)
    hh = jnp.dot(hh, l2w_ref[...], preferred_element_type=jnp.float32) + l2b_ref[...]
    m = jnp.max(hh, axis=-1, keepdims=True)
    lse = m + jnp.log(jnp.sum(jnp.exp(hh - m), axis=-1, keepdims=True))
    o_ref[...] = hh - lse


def _layer3_head(xs, agg, eps, wa, ba, wb, bb, g, be, batch, l1w, l1b, l2w,
                 l2b):
    return pl.pallas_call(
        _layer3_head_body,
        out_shape=jax.ShapeDtypeStruct((_G, _OUT), jnp.float32),
    )(xs, agg, jnp.reshape(1.0 + eps, (1, 1)), wa,
      jnp.reshape(ba, (1, _H)), wb, jnp.reshape(bb, (1, _H)),
      jnp.reshape(g, (1, _H)), jnp.reshape(be, (1, _H)),
      batch, l1w, jnp.reshape(l1b, (1, _H)), l2w, jnp.reshape(l2b, (1, _OUT)))


def kernel(x, edge_index, batch, eps1, W1a, b1a, W1b, b1b, g1, be1,
           eps2, W2a, b2a, W2b, b2b, g2, be2,
           eps3, W3a, b3a, W3b, b3b, g3, be3, l1W, l1b, l2W, l2b):
    src = edge_index[0]
    dst = edge_index[1]
    npad = _E_PAD - _E
    srcs = jnp.reshape(
        jnp.concatenate([src, jnp.zeros((npad,), jnp.int32)]),
        (_NS, _CH_PER_TEC, _CHUNK))
    dsts = jnp.reshape(
        jnp.concatenate([dst, jnp.full((npad,), _N, jnp.int32)]),
        (_NS, _CH_PER_TEC, _CHUNK))
    zeros = jnp.zeros((_ZROWS, _HD), jnp.float32)

    xp = jnp.pad(x, ((0, _ROWS - _N), (0, 0)))
    hs = jnp.stack([xp[:, :_HD], xp[:, _HD:]])

    for eps, wa, ba, wb, bb, g, be in (
            (eps1, W1a, b1a, W1b, b1b, g1, be1),
            (eps2, W2a, b2a, W2b, b2b, g2, be2)):
        agg = _sc_agg(hs, srcs, dsts, zeros)
        hs = _layer(hs, agg, eps, wa, ba, wb, bb, g, be)
    agg = _sc_agg(hs, srcs, dsts, zeros)
    return _layer3_head(hs, agg, eps3, W3a, b3a, W3b, b3b, g3, be3,
                        batch, l1W, l1b, l2W, l2b)


# speedup vs baseline: 1.0368x; 1.0368x over previous
import functools

import jax
import jax.numpy as jnp
from jax import lax
from jax.experimental import pallas as pl
from jax.experimental.pallas import tpu as pltpu
from jax.experimental.pallas import tpu_sc as plsc

_N = 10000
_E = 320000
_D = 128
_H = 128
_OUT = 10
_G = 128

_NC = 2
_NS = 16
_HD = _D // _NC
_CHUNK = 128
_CH_PER_TEC = 160
_WCH = 40
_NWIN = _CH_PER_TEC // _WCH
_E_PER_TEC = _CHUNK * _CH_PER_TEC
_E_PAD = _E_PER_TEC * _NS
_ROWS = 10240
_ZROWS = _ROWS // _NS


_NBUF = 2


def _sc_agg_body(xs_hbm, srcs_hbm, dsts_hbm, zeros_hbm, out_hbm,
                 x_sh, acc_sh, src_v, dst_v,
                 rows0_v, rows1_v, sem0, sem1):
    c = lax.axis_index("c")
    s = lax.axis_index("s")
    cp_x = pltpu.async_copy(xs_hbm.at[c, pl.ds(s * _ZROWS, _ZROWS)],
                            x_sh.at[pl.ds(s * _ZROWS, _ZROWS)], sem0)
    cp_z = pltpu.async_copy(zeros_hbm, acc_sh.at[pl.ds(s * _ZROWS, _ZROWS)],
                            sem1)
    cp_x.wait()
    cp_z.wait()
    plsc.subcore_barrier()

    rows = (rows0_v, rows1_v)
    sems = (sem0, sem1)

    def window(w, carry):
        pltpu.sync_copy(srcs_hbm.at[s, pl.ds(w * _WCH, _WCH)], src_v)
        pltpu.sync_copy(dsts_hbm.at[s, pl.ds(w * _WCH, _WCH)], dst_v)
        for b in range(_NBUF):
            pltpu.async_copy(x_sh.at[src_v.at[b]], rows[b], sems[b])

        def step(g, cy):
            for b in range(_NBUF):
                j = g * _NBUF + b
                pltpu.make_async_copy(x_sh.at[src_v.at[j]], rows[b],
                                      sems[b]).wait()
                pltpu.sync_copy(rows[b], acc_sh.at[dst_v.at[j]], add=True)
                pltpu.async_copy(x_sh.at[src_v.at[j + _NBUF]], rows[b],
                                 sems[b])
            return cy

        turns = _WCH // _NBUF - 1
        lax.fori_loop(0, turns, step, carry, unroll=False)
        for j in range(turns * _NBUF, _WCH):
            b = j % _NBUF
            pltpu.make_async_copy(x_sh.at[src_v.at[j]], rows[b],
                                  sems[b]).wait()
            pltpu.sync_copy(rows[b], acc_sh.at[dst_v.at[j]], add=True)
            if j + _NBUF < _WCH:
                pltpu.async_copy(x_sh.at[src_v.at[j + _NBUF]], rows[b],
                                 sems[b])
        return carry

    lax.fori_loop(0, _NWIN, window, 0, unroll=False)
    plsc.subcore_barrier()
    pltpu.sync_copy(acc_sh.at[pl.ds(s * _ZROWS, _ZROWS)],
                    out_hbm.at[c, pl.ds(s * _ZROWS, _ZROWS)])


@functools.cache
def _build_sc_agg():
    return functools.partial(
        pl.kernel,
        out_type=jax.ShapeDtypeStruct((_NC, _ROWS, _HD), jnp.float32),
        mesh=plsc.VectorSubcoreMesh(core_axis_name="c", subcore_axis_name="s",
                                    num_cores=_NC, num_subcores=_NS),
        scratch_types=[
            pltpu.VMEM_SHARED((_ROWS, _HD), jnp.float32),
            pltpu.VMEM_SHARED((_ROWS, _HD), jnp.float32),
            pltpu.VMEM((_WCH, _CHUNK), jnp.int32),
            pltpu.VMEM((_WCH, _CHUNK), jnp.int32),
            pltpu.VMEM((_CHUNK, _HD), jnp.float32),
            pltpu.VMEM((_CHUNK, _HD), jnp.float32),
            pltpu.SemaphoreType.DMA,
            pltpu.SemaphoreType.DMA,
        ],
    )(_sc_agg_body)


def _sc_agg(xs, srcs, dsts, zeros):
    return _build_sc_agg()(xs, srcs, dsts, zeros)


def _layer_body(xs_ref, a_ref, eps_ref, wa_ref, ba_ref, wb_ref, bb_ref,
                g_ref, be_ref, o_ref):
    x = jnp.concatenate([xs_ref[0, pl.ds(0, _N), :],
                         xs_ref[1, pl.ds(0, _N), :]], axis=1)
    agg = jnp.concatenate([a_ref[0, pl.ds(0, _N), :],
                           a_ref[1, pl.ds(0, _N), :]], axis=1)
    h = eps_ref[...] * x + agg
    h = jnp.maximum(jnp.dot(h, wa_ref[...], preferred_element_type=jnp.float32)
                    + ba_ref[...], 0.0)
    h = jnp.maximum(jnp.dot(h, wb_ref[...], preferred_element_type=jnp.float32)
                    + bb_ref[...], 0.0)
    mu = jnp.mean(h, axis=0)
    d = h - mu
    var = jnp.mean(d * d, axis=0)
    out = g_ref[...] * d * lax.rsqrt(var + 1e-5) + be_ref[...]
    o_ref[0, pl.ds(0, _N), :] = out[:, :_HD]
    o_ref[1, pl.ds(0, _N), :] = out[:, _HD:]


def _layer(xs, agg, eps, wa, ba, wb, bb, g, be):
    return pl.pallas_call(
        _layer_body,
        out_shape=jax.ShapeDtypeStruct((_NC, _ROWS, _HD), jnp.float32),
    )(xs, agg, jnp.reshape(1.0 + eps, (1, 1)), wa,
      jnp.reshape(ba, (1, _H)), wb, jnp.reshape(bb, (1, _H)),
      jnp.reshape(g, (1, _H)), jnp.reshape(be, (1, _H)))


def _layer3_head_body(xs_ref, a_ref, eps_ref, wa_ref, ba_ref, wb_ref, bb_ref,
                      g_ref, be_ref, bt_ref, l1w_ref, l1b_ref, l2w_ref,
                      l2b_ref, o_ref):
    x = jnp.concatenate([xs_ref[0, pl.ds(0, _N), :],
                         xs_ref[1, pl.ds(0, _N), :]], axis=1)
    agg = jnp.concatenate([a_ref[0, pl.ds(0, _N), :],
                           a_ref[1, pl.ds(0, _N), :]], axis=1)
    h = eps_ref[...] * x + agg
    h = jnp.maximum(jnp.dot(h, wa_ref[...], preferred_element_type=jnp.float32)
                    + ba_ref[...], 0.0)
    h = jnp.maximum(jnp.dot(h, wb_ref[...], preferred_element_type=jnp.float32)
                    + bb_ref[...], 0.0)
    mu = jnp.mean(h, axis=0)
    d = h - mu
    var = jnp.mean(d * d, axis=0)
    h = g_ref[...] * d * lax.rsqrt(var + 1e-5) + be_ref[...]
    onehot = (bt_ref[...][:, None]
              == lax.broadcasted_iota(jnp.int32, (1, _G), 1)).astype(jnp.float32)
    sums = lax.dot_general(onehot, h, (((0,), (0,)), ((), ())),
                           preferred_element_type=jnp.float32)
    cnt = jnp.sum(onehot, axis=0)
    pooled = sums / jnp.maximum(cnt, 1.0)[:, None]
    hh = jnp.maximum(jnp.dot(pooled, l1w_ref[...],
                             preferred_element_type=jnp.float32) + l1b_ref[...], 0.0)
    hh = jnp.dot(hh, l2w_ref[...], preferred_element_type=jnp.float32) + l2b_ref[...]
    m = jnp.max(hh, axis=-1, keepdims=True)
    lse = m + jnp.log(jnp.sum(jnp.exp(hh - m), axis=-1, keepdims=True))
    o_ref[...] = hh - lse


def _layer3_head(xs, agg, eps, wa, ba, wb, bb, g, be, batch, l1w, l1b, l2w,
                 l2b):
    return pl.pallas_call(
        _layer3_head_body,
        out_shape=jax.ShapeDtypeStruct((_G, _OUT), jnp.float32),
    )(xs, agg, jnp.reshape(1.0 + eps, (1, 1)), wa,
      jnp.reshape(ba, (1, _H)), wb, jnp.reshape(bb, (1, _H)),
      jnp.reshape(g, (1, _H)), jnp.reshape(be, (1, _H)),
      batch, l1w, jnp.reshape(l1b, (1, _H)), l2w, jnp.reshape(l2b, (1, _OUT)))


def kernel(x, edge_index, batch, eps1, W1a, b1a, W1b, b1b, g1, be1,
           eps2, W2a, b2a, W2b, b2b, g2, be2,
           eps3, W3a, b3a, W3b, b3b, g3, be3, l1W, l1b, l2W, l2b):
    src = edge_index[0]
    dst = edge_index[1]
    npad = _E_PAD - _E
    srcs = jnp.reshape(
        jnp.concatenate([src, jnp.zeros((npad,), jnp.int32)]),
        (_NS, _CH_PER_TEC, _CHUNK))
    dsts = jnp.reshape(
        jnp.concatenate([dst, jnp.full((npad,), _N, jnp.int32)]),
        (_NS, _CH_PER_TEC, _CHUNK))
    zeros = jnp.zeros((_ZROWS, _HD), jnp.float32)

    xp = jnp.pad(x, ((0, _ROWS - _N), (0, 0)))
    hs = jnp.stack([xp[:, :_HD], xp[:, _HD:]])

    for eps, wa, ba, wb, bb, g, be in (
            (eps1, W1a, b1a, W1b, b1b, g1, be1),
            (eps2, W2a, b2a, W2b, b2b, g2, be2)):
        agg = _sc_agg(hs, srcs, dsts, zeros)
        hs = _layer(hs, agg, eps, wa, ba, wb, bb, g, be)
    agg = _sc_agg(hs, srcs, dsts, zeros)
    return _layer3_head(hs, agg, eps3, W3a, b3a, W3b, b3b, g3, be3,
                        batch, l1W, l1b, l2W, l2b)
